# Initial kernel scaffold; baseline (speedup 1.0000x reference)
import jax
import jax.numpy as jnp
from jax.experimental import pallas as pl


def kernel(x, table):
    raise NotImplementedError("write your pallas kernel here")


# speedup vs baseline: 2.9727x; 2.9727x over previous
import functools

import jax
import jax.numpy as jnp
from jax import lax
from jax.experimental import pallas as pl
from jax.experimental.pallas import tpu as pltpu
from jax.experimental.pallas import tpu_sc as plsc

VOCAB = 100000
DIM = 128
BATCH = 4096
HIST = 50
N = BATCH * HIST
NC, NS = 2, 16
NW = NC * NS
PER_W = N // NW
CH = 128
NCH = PER_W // CH


@functools.partial(
    pl.kernel,
    mesh=plsc.VectorSubcoreMesh(core_axis_name="c", subcore_axis_name="s"),
    out_type=jax.ShapeDtypeStruct((N, DIM), jnp.float32),
    scratch_types=[
        pltpu.VMEM((NCH, CH), jnp.int32),
        pltpu.VMEM((CH, DIM), jnp.float32),
        pltpu.SemaphoreType.DMA,
    ],
)
def _emb_gather(x_hbm, table_hbm, out_hbm, idx_v, rows_v, sem):
    wid = lax.axis_index("s") * NC + lax.axis_index("c")
    pltpu.sync_copy(x_hbm.at[wid], idx_v)
    base = wid * PER_W

    def step(j, carry):
        pltpu.async_copy(table_hbm.at[idx_v.at[j]], rows_v, sem).wait()
        pltpu.sync_copy(rows_v, out_hbm.at[pl.ds(base + j * CH, CH)])
        return carry

    lax.fori_loop(0, NCH, step, 0)


def kernel(x, table):
    x2 = x.reshape(NW, NCH, CH).astype(jnp.int32)
    out = _emb_gather(x2, table)
    return out.reshape(BATCH, HIST, DIM)


# speedup vs baseline: 3.3498x; 1.1269x over previous
import functools

import jax
import jax.numpy as jnp
from jax import lax
from jax.experimental import pallas as pl
from jax.experimental.pallas import tpu as pltpu
from jax.experimental.pallas import tpu_sc as plsc

VOCAB = 100000
DIM = 128
BATCH = 4096
HIST = 50
N = BATCH * HIST
NC, NS = 2, 16
NW = NC * NS
PER_W = N // NW
CH = 128
NCH = PER_W // CH
NBUF = 5


@functools.partial(
    pl.kernel,
    mesh=plsc.VectorSubcoreMesh(core_axis_name="c", subcore_axis_name="s"),
    out_type=jax.ShapeDtypeStruct((N, DIM), jnp.float32),
    scratch_types=[
        pltpu.VMEM((NCH, CH), jnp.int32),
    ]
    + [pltpu.VMEM((CH, DIM), jnp.float32) for _ in range(NBUF)]
    + [pltpu.SemaphoreType.DMA for _ in range(2 * NBUF)],
)
def _emb_gather(x_hbm, table_hbm, out_hbm, idx_v, *bufs_and_sems):
    rows = bufs_and_sems[:NBUF]
    gsem = bufs_and_sems[NBUF:2 * NBUF]
    wsem = bufs_and_sems[2 * NBUF:]
    wid = lax.axis_index("s") * NC + lax.axis_index("c")
    pltpu.sync_copy(x_hbm.at[wid], idx_v)
    base = wid * PER_W

    for b in range(NBUF):
        pltpu.async_copy(table_hbm.at[idx_v.at[b]], rows[b], gsem[b])

    def outer(t, carry):
        for b in range(NBUF):
            j = t * NBUF + b
            pltpu.make_async_copy(
                table_hbm.at[pl.ds(0, CH)], rows[b], gsem[b]).wait()
            pltpu.async_copy(
                rows[b], out_hbm.at[pl.ds(base + j * CH, CH)], wsem[b])
            pltpu.make_async_copy(
                rows[b], out_hbm.at[pl.ds(0, CH)], wsem[b]).wait()
            pltpu.async_copy(
                table_hbm.at[idx_v.at[j + NBUF]], rows[b], gsem[b])
        return carry

    lax.fori_loop(0, NCH // NBUF - 1, outer, 0)

    for b in range(NBUF):
        j = NCH - NBUF + b
        pltpu.make_async_copy(
            table_hbm.at[pl.ds(0, CH)], rows[b], gsem[b]).wait()
        pltpu.sync_copy(rows[b], out_hbm.at[pl.ds(base + j * CH, CH)])


def kernel(x, table):
    x2 = x.reshape(NW, NCH, CH).astype(jnp.int32)
    out = _emb_gather(x2, table)
    return out.reshape(BATCH, HIST, DIM)


# speedup vs baseline: 5.9246x; 1.7687x over previous
import functools

import jax
import jax.numpy as jnp
from jax import lax
from jax.experimental import pallas as pl
from jax.experimental.pallas import tpu as pltpu
from jax.experimental.pallas import tpu_sc as plsc

VOCAB = 100000
DIM = 128
BATCH = 4096
HIST = 50
NC, NS = 2, 16
NW = NC * NS
SPW = BATCH // NW
G = 8
NCH = SPW // G
NBUF = 2


@functools.partial(
    pl.kernel,
    mesh=plsc.VectorSubcoreMesh(core_axis_name="c", subcore_axis_name="s"),
    out_type=jax.ShapeDtypeStruct((BATCH, HIST, DIM), jnp.float32),
    scratch_types=[
        pltpu.VMEM((SPW, HIST), jnp.int32),
    ]
    + [pltpu.VMEM((G, HIST, DIM), jnp.float32) for _ in range(NBUF)]
    + [pltpu.SemaphoreType.DMA for _ in range(2 * NBUF)],
)
def _emb_gather(x_hbm, table_hbm, out_hbm, idx_v, *bufs_and_sems):
    bufs = bufs_and_sems[:NBUF]
    gsem = bufs_and_sems[NBUF:2 * NBUF]
    wsem = bufs_and_sems[2 * NBUF:]
    wid = lax.axis_index("s") * NC + lax.axis_index("c")
    base = wid * SPW
    pltpu.sync_copy(x_hbm.at[pl.ds(base, SPW)], idx_v)

    def fire_gathers(c, b):
        for g in range(G):
            pltpu.async_copy(
                table_hbm.at[idx_v.at[c * G + g]], bufs[b].at[g], gsem[b])

    def wait_gathers(b):
        for g in range(G):
            pltpu.make_async_copy(
                out_hbm.at[0], bufs[b].at[g], gsem[b]).wait()

    for b in range(NBUF):
        fire_gathers(b, b)

    def outer(t, carry):
        for b in range(NBUF):
            c = t * NBUF + b
            wait_gathers(b)
            pltpu.async_copy(
                bufs[b], out_hbm.at[pl.ds(base + c * G, G)], wsem[b])
            pltpu.make_async_copy(
                bufs[b], out_hbm.at[pl.ds(0, G)], wsem[b]).wait()
            fire_gathers(c + NBUF, b)
        return carry

    lax.fori_loop(0, NCH // NBUF - 1, outer, 0)

    for b in range(NBUF):
        c = NCH - NBUF + b
        wait_gathers(b)
        pltpu.sync_copy(bufs[b], out_hbm.at[pl.ds(base + c * G, G)])


def kernel(x, table):
    return _emb_gather(x.astype(jnp.int32), table)
